# baseline (speedup 1.0000x reference)
import jax
import jax.numpy as jnp
from jax import lax
from jax.experimental import pallas as pl
from jax.experimental.pallas import tpu as pltpu
from jax.experimental.pallas import tpu_sc as plsc

BATCH = 16384
DIM = 32
NUM_CORES = 2
NUM_SUBCORES = 16
NUM_WORKERS = NUM_CORES * NUM_SUBCORES
LANES = 16

B_SC = 8192
B_TC = BATCH - B_SC
B_PER_W = B_SC // NUM_WORKERS


def _sc_body(u_hbm, i_hbm, uw_hbm, iw_hbm, dummy_hbm, out_hbm,
             u_idx, i_idx, ue_rows, ie_rows, out_v, sem):
    wid = lax.axis_index("s") * NUM_CORES + lax.axis_index("c")

    pltpu.sync_copy(u_hbm.at[pl.ds(wid * B_PER_W, B_PER_W)], u_idx)
    pltpu.sync_copy(i_hbm.at[pl.ds(wid * B_PER_W, B_PER_W)], i_idx)

    lane_iota = lax.iota(jnp.int32, LANES)

    @pl.loop(0, B_PER_W // LANES)
    def _fire(b):
        base = b * LANES
        uv = u_idx[pl.ds(base, LANES)]
        iv = i_idx[pl.ds(base, LANES)]
        for l in range(LANES):
            dst = base + l
            pltpu.async_copy(
                uw_hbm.at[pl.ds(uv[l], 1)],
                ue_rows.at[pl.ds(dst, 1)], sem)
            pltpu.async_copy(
                iw_hbm.at[pl.ds(iv[l], 1)],
                ie_rows.at[pl.ds(dst, 1)], sem)

    pltpu.make_async_copy(dummy_hbm, ue_rows, sem).wait()
    pltpu.make_async_copy(dummy_hbm, ie_rows, sem).wait()

    @pl.loop(0, B_PER_W // LANES)
    def _group(g):
        vrow = g * LANES + lane_iota
        acc = jnp.zeros((LANES,), jnp.float32)
        for k in range(DIM):
            vcol = jnp.full((LANES,), k, jnp.int32)
            gu = plsc.load_gather(ue_rows, [vrow, vcol])
            gi = plsc.load_gather(ie_rows, [vrow, vcol])
            acc = acc + gu * gi
        out_v[pl.ds(g * LANES, LANES)] = acc

    pltpu.sync_copy(out_v, out_hbm.at[pl.ds(wid * B_PER_W, B_PER_W)])


def _run_sc(u_sc, i_sc, user_weight, item_weight):
    dummy = jnp.zeros((B_PER_W, DIM), jnp.float32)
    mesh = plsc.VectorSubcoreMesh(
        core_axis_name="c", subcore_axis_name="s",
        num_cores=NUM_CORES, num_subcores=NUM_SUBCORES)
    run = pl.kernel(
        _sc_body,
        out_type=jax.ShapeDtypeStruct((B_SC,), jnp.float32),
        mesh=mesh,
        compiler_params=pltpu.CompilerParams(needs_layout_passes=False,
                                             use_tc_tiling_on_sc=True),
        scratch_types=[
            pltpu.VMEM((B_PER_W,), jnp.int32),
            pltpu.VMEM((B_PER_W,), jnp.int32),
            pltpu.VMEM((B_PER_W, DIM), jnp.float32),
            pltpu.VMEM((B_PER_W, DIM), jnp.float32),
            pltpu.VMEM((B_PER_W,), jnp.float32),
            pltpu.SemaphoreType.DMA,
        ],
    )
    return run(u_sc, i_sc, user_weight, item_weight, dummy)


def _tc_body(u_smem, i_smem, uw_any, iw_any, dummy_any, out_vmem,
             ue_vmem, ie_vmem, sem):
    def fire(r, _):
        pltpu.make_async_copy(
            uw_any.at[pl.ds(u_smem[r], 1)],
            ue_vmem.at[pl.ds(r, 1)], sem).start()
        pltpu.make_async_copy(
            iw_any.at[pl.ds(i_smem[r], 1)],
            ie_vmem.at[pl.ds(r, 1)], sem).start()
        return ()

    lax.fori_loop(0, B_TC, fire, (), unroll=8)

    pltpu.make_async_copy(dummy_any, ue_vmem, sem).wait()
    pltpu.make_async_copy(dummy_any, ie_vmem, sem).wait()

    out_vmem[...] = jnp.sum(ue_vmem[...] * ie_vmem[...], axis=1)


def _run_tc(u_tc, i_tc, user_weight, item_weight):
    dummy = jnp.zeros((B_TC, DIM), jnp.float32)
    return pl.pallas_call(
        _tc_body,
        out_shape=jax.ShapeDtypeStruct((B_TC,), jnp.float32),
        in_specs=[
            pl.BlockSpec(memory_space=pltpu.SMEM),
            pl.BlockSpec(memory_space=pltpu.SMEM),
            pl.BlockSpec(memory_space=pl.ANY),
            pl.BlockSpec(memory_space=pl.ANY),
            pl.BlockSpec(memory_space=pl.ANY),
        ],
        scratch_shapes=[
            pltpu.VMEM((B_TC, DIM), jnp.float32),
            pltpu.VMEM((B_TC, DIM), jnp.float32),
            pltpu.SemaphoreType.DMA,
        ],
    )(u_tc, i_tc, user_weight, item_weight, dummy)


def kernel(u, i, user_weight, item_weight):
    u2 = u.astype(jnp.int32)
    i2 = i.astype(jnp.int32)
    out_tc = _run_tc(u2[B_SC:], i2[B_SC:], user_weight, item_weight)
    out_tc2 = _run_tc(u2[:B_SC], i2[:B_SC], user_weight, item_weight)
    return jnp.concatenate([out_tc2, out_tc])


# speedup vs baseline: 1.0907x; 1.0907x over previous
import jax
import jax.numpy as jnp
from jax import lax
from jax.experimental import pallas as pl
from jax.experimental.pallas import tpu as pltpu
from jax.experimental.pallas import tpu_sc as plsc

BATCH = 16384
DIM = 32
NUM_CORES = 2
NUM_SUBCORES = 16
NUM_WORKERS = NUM_CORES * NUM_SUBCORES
LANES = 16

B_SC = 8192
B_TC = BATCH - B_SC
B_PER_W = B_SC // NUM_WORKERS


def _sc_body(u_hbm, i_hbm, uw_hbm, iw_hbm, dummy_hbm, out_hbm,
             u_idx, i_idx, ue_rows, ie_rows, out_v, sem):
    wid = lax.axis_index("s") * NUM_CORES + lax.axis_index("c")

    pltpu.sync_copy(u_hbm.at[pl.ds(wid * B_PER_W, B_PER_W)], u_idx)
    pltpu.sync_copy(i_hbm.at[pl.ds(wid * B_PER_W, B_PER_W)], i_idx)

    lane_iota = lax.iota(jnp.int32, LANES)

    @pl.loop(0, B_PER_W // LANES)
    def _fire(b):
        base = b * LANES
        uv = u_idx[pl.ds(base, LANES)]
        iv = i_idx[pl.ds(base, LANES)]
        for l in range(LANES):
            dst = base + l
            pltpu.async_copy(
                uw_hbm.at[pl.ds(uv[l], 1)],
                ue_rows.at[pl.ds(dst, 1)], sem)
            pltpu.async_copy(
                iw_hbm.at[pl.ds(iv[l], 1)],
                ie_rows.at[pl.ds(dst, 1)], sem)

    pltpu.make_async_copy(dummy_hbm, ue_rows, sem).wait()
    pltpu.make_async_copy(dummy_hbm, ie_rows, sem).wait()

    @pl.loop(0, B_PER_W // LANES)
    def _group(g):
        vrow = g * LANES + lane_iota
        acc = jnp.zeros((LANES,), jnp.float32)
        for k in range(DIM):
            vcol = jnp.full((LANES,), k, jnp.int32)
            gu = plsc.load_gather(ue_rows, [vrow, vcol])
            gi = plsc.load_gather(ie_rows, [vrow, vcol])
            acc = acc + gu * gi
        out_v[pl.ds(g * LANES, LANES)] = acc

    pltpu.sync_copy(out_v, out_hbm.at[pl.ds(wid * B_PER_W, B_PER_W)])


def _run_sc(u_sc, i_sc, user_weight, item_weight):
    dummy = jnp.zeros((B_PER_W, DIM), jnp.float32)
    mesh = plsc.VectorSubcoreMesh(
        core_axis_name="c", subcore_axis_name="s",
        num_cores=NUM_CORES, num_subcores=NUM_SUBCORES)
    run = pl.kernel(
        _sc_body,
        out_type=jax.ShapeDtypeStruct((B_SC,), jnp.float32),
        mesh=mesh,
        compiler_params=pltpu.CompilerParams(needs_layout_passes=False,
                                             use_tc_tiling_on_sc=True),
        cost_estimate=pl.CostEstimate(flops=0, transcendentals=0,
                                      bytes_accessed=1_000_000_000),
        scratch_types=[
            pltpu.VMEM((B_PER_W,), jnp.int32),
            pltpu.VMEM((B_PER_W,), jnp.int32),
            pltpu.VMEM((B_PER_W, DIM), jnp.float32),
            pltpu.VMEM((B_PER_W, DIM), jnp.float32),
            pltpu.VMEM((B_PER_W,), jnp.float32),
            pltpu.SemaphoreType.DMA,
        ],
    )
    return run(u_sc, i_sc, user_weight, item_weight, dummy)


def _tc_body(u_smem, i_smem, uw_any, iw_any, dummy_any, out_vmem,
             ue_vmem, ie_vmem, sem):
    def fire(r, _):
        pltpu.make_async_copy(
            uw_any.at[pl.ds(u_smem[r], 1)],
            ue_vmem.at[pl.ds(r, 1)], sem).start()
        pltpu.make_async_copy(
            iw_any.at[pl.ds(i_smem[r], 1)],
            ie_vmem.at[pl.ds(r, 1)], sem).start()
        return ()

    lax.fori_loop(0, B_TC, fire, (), unroll=8)

    pltpu.make_async_copy(dummy_any, ue_vmem, sem).wait()
    pltpu.make_async_copy(dummy_any, ie_vmem, sem).wait()

    out_vmem[...] = jnp.sum(ue_vmem[...] * ie_vmem[...], axis=1)


def _run_tc(u_tc, i_tc, user_weight, item_weight):
    dummy = jnp.zeros((B_TC, DIM), jnp.float32)
    return pl.pallas_call(
        _tc_body,
        out_shape=jax.ShapeDtypeStruct((B_TC,), jnp.float32),
        in_specs=[
            pl.BlockSpec(memory_space=pltpu.SMEM),
            pl.BlockSpec(memory_space=pltpu.SMEM),
            pl.BlockSpec(memory_space=pl.ANY),
            pl.BlockSpec(memory_space=pl.ANY),
            pl.BlockSpec(memory_space=pl.ANY),
        ],
        scratch_shapes=[
            pltpu.VMEM((B_TC, DIM), jnp.float32),
            pltpu.VMEM((B_TC, DIM), jnp.float32),
            pltpu.SemaphoreType.DMA,
        ],
    )(u_tc, i_tc, user_weight, item_weight, dummy)


def kernel(u, i, user_weight, item_weight):
    u2 = u.astype(jnp.int32)
    i2 = i.astype(jnp.int32)
    out_sc = _run_sc(u2[:B_SC], i2[:B_SC], user_weight, item_weight)
    out_tc = _run_tc(u2[B_SC:], i2[B_SC:], user_weight, item_weight)
    return jnp.concatenate([out_sc, out_tc])
